# Initial kernel scaffold; baseline (speedup 1.0000x reference)
import jax
import jax.numpy as jnp
from jax.experimental import pallas as pl


def kernel(h, edge_index, edge_attr, We1, be1, We2, be2, Wn1, bn1, Wn2, bn2):
    raise NotImplementedError("write your pallas kernel here")


# speedup vs baseline: 2.5099x; 2.5099x over previous
import functools

import jax
import jax.numpy as jnp
from jax import lax
from jax.experimental import pallas as pl
from jax.experimental.pallas import tpu as pltpu
from jax.experimental.pallas import tpu_sc as plsc

N = 10000
E = 160000
D = 256
H = 256
DE = 16

NC = 2
NS = 16
NW = NC * NS


_PRE_TILE = 2000


def _pre_body(h_ref, ws_ref, wt_ref, a_ref, b_ref):
    hb = h_ref[...]
    a_ref[...] = jnp.dot(hb, ws_ref[...], preferred_element_type=jnp.float32)
    b_ref[...] = jnp.dot(hb, wt_ref[...], preferred_element_type=jnp.float32)


def _pre_pass(h, ws, wt):
    grid = (N // _PRE_TILE,)
    return pl.pallas_call(
        _pre_body,
        grid=grid,
        in_specs=[
            pl.BlockSpec((_PRE_TILE, D), lambda i: (i, 0)),
            pl.BlockSpec((D, H), lambda i: (0, 0)),
            pl.BlockSpec((D, H), lambda i: (0, 0)),
        ],
        out_specs=[
            pl.BlockSpec((_PRE_TILE, H), lambda i: (i, 0)),
            pl.BlockSpec((_PRE_TILE, H), lambda i: (i, 0)),
        ],
        out_shape=[
            jax.ShapeDtypeStruct((N, H), jnp.float32),
            jax.ShapeDtypeStruct((N, H), jnp.float32),
        ],
    )(h, ws, wt)


_GCH = 128
_EPW = E // NW
_GFULL = _EPW // _GCH
_GTAIL = _EPW - _GFULL * _GCH


def _gather_add_rows(buf_a, buf_b, n_rows):
    def body(i, _):
        for j in range(H // 16):
            sl = pl.ds(j * 16, 16)
            plsc.addupdate(buf_a.at[i, sl], buf_b[i, sl])
        return 0

    lax.fori_loop(0, n_rows, body, 0, unroll=False)


def _gather_kernel(a_hbm, b_hbm, row_hbm, col_hbm, g_hbm,
                   idx_r, idx_c, buf_a, buf_b,
                   idx_r8, idx_c8, buf_a8, buf_b8,
                   sem_a, sem_b):
    c = lax.axis_index("c")
    s = lax.axis_index("s")
    wid = c * NS + s
    base = wid * _EPW

    def chunk(k, _):
        off = base + k * _GCH
        pltpu.sync_copy(row_hbm.at[pl.ds(off, _GCH)], idx_r)
        pltpu.sync_copy(col_hbm.at[pl.ds(off, _GCH)], idx_c)
        cp_a = pltpu.async_copy(a_hbm.at[idx_r], buf_a, sem_a)
        cp_b = pltpu.async_copy(b_hbm.at[idx_c], buf_b, sem_b)
        cp_a.wait()
        cp_b.wait()
        _gather_add_rows(buf_a, buf_b, _GCH)
        pltpu.sync_copy(buf_a, g_hbm.at[pl.ds(off, _GCH)])
        return 0

    lax.fori_loop(0, _GFULL, chunk, 0, unroll=False)

    off = base + _GFULL * _GCH
    pltpu.sync_copy(row_hbm.at[pl.ds(off, _GTAIL)], idx_r8)
    pltpu.sync_copy(col_hbm.at[pl.ds(off, _GTAIL)], idx_c8)
    cp_a = pltpu.async_copy(a_hbm.at[idx_r8], buf_a8, sem_a)
    cp_b = pltpu.async_copy(b_hbm.at[idx_c8], buf_b8, sem_b)
    cp_a.wait()
    cp_b.wait()
    _gather_add_rows(buf_a8, buf_b8, _GTAIL)
    pltpu.sync_copy(buf_a8, g_hbm.at[pl.ds(off, _GTAIL)])


def _gather(a, b, row, col):
    mesh = plsc.VectorSubcoreMesh(core_axis_name="c", subcore_axis_name="s")
    f = pl.kernel(
        _gather_kernel,
        out_type=jax.ShapeDtypeStruct((E, H), jnp.float32),
        mesh=mesh,
        scratch_types=[
            pltpu.VMEM((_GCH,), jnp.int32),
            pltpu.VMEM((_GCH,), jnp.int32),
            pltpu.VMEM((_GCH, H), jnp.float32),
            pltpu.VMEM((_GCH, H), jnp.float32),
            pltpu.VMEM((_GTAIL,), jnp.int32),
            pltpu.VMEM((_GTAIL,), jnp.int32),
            pltpu.VMEM((_GTAIL, H), jnp.float32),
            pltpu.VMEM((_GTAIL, H), jnp.float32),
            pltpu.SemaphoreType.DMA,
            pltpu.SemaphoreType.DMA,
        ],
    )
    return f(a, b, row, col)


_EDGE_TILE = 2000


def _edge_body(g_ref, ea_ref, wea_ref, be1_ref, we2_ref, be2_ref, out_ref):
    pre1 = (g_ref[...]
            + jnp.dot(ea_ref[...], wea_ref[...],
                      preferred_element_type=jnp.float32)
            + be1_ref[...])
    m = pre1 * jax.nn.sigmoid(pre1)
    pre2 = jnp.dot(m, we2_ref[...], preferred_element_type=jnp.float32) \
        + be2_ref[...]
    out_ref[...] = pre2 * jax.nn.sigmoid(pre2)


def _edge_mlp(g, ea, wea, be1, we2, be2):
    grid = (E // _EDGE_TILE,)
    return pl.pallas_call(
        _edge_body,
        grid=grid,
        in_specs=[
            pl.BlockSpec((_EDGE_TILE, H), lambda i: (i, 0)),
            pl.BlockSpec((_EDGE_TILE, DE), lambda i: (i, 0)),
            pl.BlockSpec((DE, H), lambda i: (0, 0)),
            pl.BlockSpec((1, H), lambda i: (0, 0)),
            pl.BlockSpec((H, H), lambda i: (0, 0)),
            pl.BlockSpec((1, H), lambda i: (0, 0)),
        ],
        out_specs=pl.BlockSpec((_EDGE_TILE, H), lambda i: (i, 0)),
        out_shape=jax.ShapeDtypeStruct((E, H), jnp.float32),
    )(g, ea, wea, be1, we2, be2)


_SCH = 128
_EPS = E // NS
_SFULL = _EPS // _SCH
_STAIL = _EPS - _SFULL * _SCH
_HH = H // NC
_RPS = 624
_RTAIL = N - NS * _RPS
_ZROWS = 208


def _scatter_kernel(mij_hbm, row_hbm, agg_hbm,
                    idx_s, buf_s, idx_t, buf_t, zbuf, acc):
    c = lax.axis_index("c")
    s = lax.axis_index("s")

    def zrow(i, _):
        for j in range(_HH // 16):
            zbuf[i, pl.ds(j * 16, 16)] = jnp.zeros((16,), jnp.float32)
        return 0

    lax.fori_loop(0, _ZROWS, zrow, 0, unroll=False)
    for k in range(_RPS // _ZROWS):
        pltpu.sync_copy(zbuf, acc.at[pl.ds(s * _RPS + k * _ZROWS, _ZROWS)])

    @pl.when(s == NS - 1)
    def _zero_tail():
        pltpu.sync_copy(zbuf.at[pl.ds(0, _RTAIL)],
                        acc.at[pl.ds(NS * _RPS, _RTAIL)])

    plsc.subcore_barrier()

    base = s * _EPS

    def chunk(k, _):
        off = base + k * _SCH
        pltpu.sync_copy(row_hbm.at[pl.ds(off, _SCH)], idx_s)
        pltpu.sync_copy(mij_hbm.at[pl.ds(off, _SCH), pl.ds(c * _HH, _HH)],
                        buf_s)
        pltpu.sync_copy(buf_s, acc.at[idx_s], add=True)
        return 0

    lax.fori_loop(0, _SFULL, chunk, 0, unroll=False)

    off = base + _SFULL * _SCH
    pltpu.sync_copy(row_hbm.at[pl.ds(off, _STAIL)], idx_t)
    pltpu.sync_copy(mij_hbm.at[pl.ds(off, _STAIL), pl.ds(c * _HH, _HH)], buf_t)
    pltpu.sync_copy(buf_t, acc.at[idx_t], add=True)

    plsc.subcore_barrier()

    r0 = s * _RPS
    pltpu.sync_copy(acc.at[pl.ds(r0, _RPS)], agg_hbm.at[c, pl.ds(r0, _RPS)])

    @pl.when(s == NS - 1)
    def _write_tail():
        pltpu.sync_copy(acc.at[pl.ds(NS * _RPS, _RTAIL)],
                        agg_hbm.at[c, pl.ds(NS * _RPS, _RTAIL)])


def _scatter(mij, row):
    mesh = plsc.VectorSubcoreMesh(core_axis_name="c", subcore_axis_name="s")
    f = pl.kernel(
        _scatter_kernel,
        out_type=jax.ShapeDtypeStruct((NC, N, _HH), jnp.float32),
        mesh=mesh,
        scratch_types=[
            pltpu.VMEM((_SCH,), jnp.int32),
            pltpu.VMEM((_SCH, _HH), jnp.float32),
            pltpu.VMEM((_STAIL,), jnp.int32),
            pltpu.VMEM((_STAIL, _HH), jnp.float32),
            pltpu.VMEM((_ZROWS, _HH), jnp.float32),
            pltpu.VMEM_SHARED((N, _HH), jnp.float32),
        ],
    )
    return f(mij, row)


_NODE_TILE = 2000


def _node_body(h_ref, a0_ref, a1_ref, wa_ref, w0_ref, w1_ref, bn1_ref,
               wn2_ref, bn2_ref, out_ref):
    hb = h_ref[...]
    pre = (jnp.dot(hb, wa_ref[...], preferred_element_type=jnp.float32)
           + jnp.dot(a0_ref[...], w0_ref[...],
                     preferred_element_type=jnp.float32)
           + jnp.dot(a1_ref[...], w1_ref[...],
                     preferred_element_type=jnp.float32)
           + bn1_ref[...])
    y = pre * jax.nn.sigmoid(pre)
    out_ref[...] = hb + jnp.dot(y, wn2_ref[...],
                                preferred_element_type=jnp.float32) \
        + bn2_ref[...]


def _node_mlp(h, a0, a1, wa, w0, w1, bn1, wn2, bn2):
    grid = (N // _NODE_TILE,)
    return pl.pallas_call(
        _node_body,
        grid=grid,
        in_specs=[
            pl.BlockSpec((_NODE_TILE, D), lambda i: (i, 0)),
            pl.BlockSpec((_NODE_TILE, _HH), lambda i: (i, 0)),
            pl.BlockSpec((_NODE_TILE, _HH), lambda i: (i, 0)),
            pl.BlockSpec((D, H), lambda i: (0, 0)),
            pl.BlockSpec((_HH, H), lambda i: (0, 0)),
            pl.BlockSpec((_HH, H), lambda i: (0, 0)),
            pl.BlockSpec((1, H), lambda i: (0, 0)),
            pl.BlockSpec((H, D), lambda i: (0, 0)),
            pl.BlockSpec((1, D), lambda i: (0, 0)),
        ],
        out_specs=pl.BlockSpec((_NODE_TILE, D), lambda i: (i, 0)),
        out_shape=jax.ShapeDtypeStruct((N, D), jnp.float32),
    )(h, a0, a1, wa, w0, w1, bn1, wn2, bn2)


def kernel(h, edge_index, edge_attr, We1, be1, We2, be2, Wn1, bn1, Wn2, bn2):
    row = edge_index[0]
    col = edge_index[1]
    ws = We1[:D]
    wt = We1[D:2 * D]
    wea = We1[2 * D:]

    a, b = _pre_pass(h, ws, wt)
    g = _gather(a, b, row, col)
    mij = _edge_mlp(g, edge_attr, wea, be1.reshape(1, H), We2,
                    be2.reshape(1, H))
    agg = _scatter(mij, row)
    h_out = _node_mlp(h, agg[0], agg[1], Wn1[:D], Wn1[D:D + _HH],
                      Wn1[D + _HH:], bn1.reshape(1, H), Wn2,
                      bn2.reshape(1, D))
    return (h_out, mij)


# speedup vs baseline: 3.6678x; 1.4613x over previous
import jax
import jax.numpy as jnp
from jax import lax
from jax.experimental import pallas as pl
from jax.experimental.pallas import tpu as pltpu
from jax.experimental.pallas import tpu_sc as plsc

N = 10000
E = 160000
D = 256
H = 256
DE = 16

NC = 2
NS = 16
NW = NC * NS

_HW = H // 2
_GCH = 104

_EA = 80128
_EB = E - _EA


_PRE_TILE = 2000


def _pack_halves(x):
    xb = x.astype(jnp.bfloat16)
    lo = lax.bitcast_convert_type(xb[:, :_HW], jnp.uint16).astype(jnp.uint32)
    hi = lax.bitcast_convert_type(xb[:, _HW:], jnp.uint16).astype(jnp.uint32)
    return lax.bitcast_convert_type((hi << 16) | lo, jnp.int32)


def _pre_body(h_ref, ws_ref, wt_ref, a_ref, b_ref):
    hb = h_ref[...]
    a_ref[...] = _pack_halves(
        jnp.dot(hb, ws_ref[...], preferred_element_type=jnp.float32))
    b_ref[...] = _pack_halves(
        jnp.dot(hb, wt_ref[...], preferred_element_type=jnp.float32))


def _pre_pass(h, ws, wt):
    grid = (N // _PRE_TILE,)
    return pl.pallas_call(
        _pre_body,
        grid=grid,
        in_specs=[
            pl.BlockSpec((_PRE_TILE, D), lambda i: (i, 0)),
            pl.BlockSpec((D, H), lambda i: (0, 0)),
            pl.BlockSpec((D, H), lambda i: (0, 0)),
        ],
        out_specs=[
            pl.BlockSpec((_PRE_TILE, _HW), lambda i: (i, 0)),
            pl.BlockSpec((_PRE_TILE, _HW), lambda i: (i, 0)),
        ],
        out_shape=[
            jax.ShapeDtypeStruct((N, _HW), jnp.int32),
            jax.ShapeDtypeStruct((N, _HW), jnp.int32),
        ],
    )(h, ws, wt)


def _make_gather(ec):
    epw = ec // NW
    gfull = epw // _GCH
    gpair = gfull // 2
    gtail = epw - gfull * _GCH

    def body(a_hbm, b_hbm, row_hbm, col_hbm, g1_hbm, g2_hbm,
             idx_r, idx_c, buf_a0, buf_b0, buf_a1, buf_b1,
             idx_rt, idx_ct, buf_at, buf_bt,
             sem_i, sem_a0, sem_b0, sem_a1, sem_b1, sem_w0, sem_w1):
        c = lax.axis_index("c")
        s = lax.axis_index("s")
        wid = c * NS + s
        base = wid * epw

        def pair(t, _):
            off0 = base + (2 * t) * _GCH
            off1 = off0 + _GCH
            ci_r = pltpu.async_copy(row_hbm.at[pl.ds(off0, 2 * _GCH)], idx_r,
                                    sem_i)
            ci_c = pltpu.async_copy(col_hbm.at[pl.ds(off0, 2 * _GCH)], idx_c,
                                    sem_i)
            ci_r.wait()
            ci_c.wait()

            @pl.when(t > 0)
            def _drain0():
                prev = off0 - 2 * _GCH
                pltpu.make_async_copy(
                    buf_a0, g1_hbm.at[pl.ds(prev, _GCH)], sem_w0).wait()
                pltpu.make_async_copy(
                    buf_b0, g2_hbm.at[pl.ds(prev, _GCH)], sem_w0).wait()

            cp_a0 = pltpu.async_copy(a_hbm.at[idx_r.at[pl.ds(0, _GCH)]],
                                     buf_a0, sem_a0)
            cp_b0 = pltpu.async_copy(b_hbm.at[idx_c.at[pl.ds(0, _GCH)]],
                                     buf_b0, sem_b0)

            @pl.when(t > 0)
            def _drain1():
                prev = off1 - 2 * _GCH
                pltpu.make_async_copy(
                    buf_a1, g1_hbm.at[pl.ds(prev, _GCH)], sem_w1).wait()
                pltpu.make_async_copy(
                    buf_b1, g2_hbm.at[pl.ds(prev, _GCH)], sem_w1).wait()

            cp_a1 = pltpu.async_copy(a_hbm.at[idx_r.at[pl.ds(_GCH, _GCH)]],
                                     buf_a1, sem_a1)
            cp_b1 = pltpu.async_copy(b_hbm.at[idx_c.at[pl.ds(_GCH, _GCH)]],
                                     buf_b1, sem_b1)

            cp_a0.wait()
            cp_b0.wait()
            pltpu.async_copy(buf_a0, g1_hbm.at[pl.ds(off0, _GCH)], sem_w0)
            pltpu.async_copy(buf_b0, g2_hbm.at[pl.ds(off0, _GCH)], sem_w0)

            cp_a1.wait()
            cp_b1.wait()
            pltpu.async_copy(buf_a1, g1_hbm.at[pl.ds(off1, _GCH)], sem_w1)
            pltpu.async_copy(buf_b1, g2_hbm.at[pl.ds(off1, _GCH)], sem_w1)
            return 0

        lax.fori_loop(0, gpair, pair, 0, unroll=False)

        last0 = base + (gfull - 2) * _GCH
        last1 = last0 + _GCH
        pltpu.make_async_copy(buf_a0, g1_hbm.at[pl.ds(last0, _GCH)],
                              sem_w0).wait()
        pltpu.make_async_copy(buf_b0, g2_hbm.at[pl.ds(last0, _GCH)],
                              sem_w0).wait()
        pltpu.make_async_copy(buf_a1, g1_hbm.at[pl.ds(last1, _GCH)],
                              sem_w1).wait()
        pltpu.make_async_copy(buf_b1, g2_hbm.at[pl.ds(last1, _GCH)],
                              sem_w1).wait()

        if gtail:
            off = base + gfull * _GCH
            pltpu.sync_copy(row_hbm.at[pl.ds(off, gtail)], idx_rt)
            pltpu.sync_copy(col_hbm.at[pl.ds(off, gtail)], idx_ct)
            cp_a = pltpu.async_copy(a_hbm.at[idx_rt], buf_at, sem_a0)
            cp_b = pltpu.async_copy(b_hbm.at[idx_ct], buf_bt, sem_b0)
            cp_a.wait()
            cp_b.wait()
            pltpu.sync_copy(buf_at, g1_hbm.at[pl.ds(off, gtail)])
            pltpu.sync_copy(buf_bt, g2_hbm.at[pl.ds(off, gtail)])

    tl = max(gtail, 8)
    mesh = plsc.VectorSubcoreMesh(core_axis_name="c", subcore_axis_name="s")
    return pl.kernel(
        body,
        out_type=[
            jax.ShapeDtypeStruct((ec, _HW), jnp.int32),
            jax.ShapeDtypeStruct((ec, _HW), jnp.int32),
        ],
        mesh=mesh,
        scratch_types=[
            pltpu.VMEM((2 * _GCH,), jnp.int32),
            pltpu.VMEM((2 * _GCH,), jnp.int32),
            pltpu.VMEM((_GCH, _HW), jnp.int32),
            pltpu.VMEM((_GCH, _HW), jnp.int32),
            pltpu.VMEM((_GCH, _HW), jnp.int32),
            pltpu.VMEM((_GCH, _HW), jnp.int32),
            pltpu.VMEM((tl,), jnp.int32),
            pltpu.VMEM((tl,), jnp.int32),
            pltpu.VMEM((tl, _HW), jnp.int32),
            pltpu.VMEM((tl, _HW), jnp.int32),
            pltpu.SemaphoreType.DMA,
            pltpu.SemaphoreType.DMA,
            pltpu.SemaphoreType.DMA,
            pltpu.SemaphoreType.DMA,
            pltpu.SemaphoreType.DMA,
            pltpu.SemaphoreType.DMA,
            pltpu.SemaphoreType.DMA,
        ],
    )


def _gather_a(*args):
    return _make_gather(_EA)(*args)


def _gather_b(*args):
    return _make_gather(_EB)(*args)


def _unpack_halves(g32):
    gu = lax.bitcast_convert_type(g32, jnp.uint32)
    lo = lax.bitcast_convert_type((gu & 0xFFFF).astype(jnp.uint16),
                                  jnp.bfloat16).astype(jnp.float32)
    hi = lax.bitcast_convert_type((gu >> 16).astype(jnp.uint16),
                                  jnp.bfloat16).astype(jnp.float32)
    return lo, hi


def _edge_body(g1_ref, g2_ref, ea_ref, wea_ref, be1_ref, we2_ref, be2_ref,
               out_ref):
    g1_lo, g1_hi = _unpack_halves(g1_ref[...])
    g2_lo, g2_hi = _unpack_halves(g2_ref[...])
    ea = ea_ref[...]
    eam = jnp.dot(ea, wea_ref[...], preferred_element_type=jnp.float32)
    pre_lo = g1_lo + g2_lo + eam[:, :_HW] + be1_ref[:, :_HW]
    pre_hi = g1_hi + g2_hi + eam[:, _HW:] + be1_ref[:, _HW:]
    m_lo = pre_lo * jax.nn.sigmoid(pre_lo)
    m_hi = pre_hi * jax.nn.sigmoid(pre_hi)
    pre2 = (jnp.dot(m_lo.astype(jnp.bfloat16), we2_ref[:_HW],
                    preferred_element_type=jnp.float32)
            + jnp.dot(m_hi.astype(jnp.bfloat16), we2_ref[_HW:],
                      preferred_element_type=jnp.float32)
            + be2_ref[...])
    out_ref[...] = pre2 * jax.nn.sigmoid(pre2)


def _edge_mlp(g1, g2, ea, wea, be1, we2, be2):
    ec = g1.shape[0]
    tile = ec // 32
    return pl.pallas_call(
        _edge_body,
        grid=(32,),
        in_specs=[
            pl.BlockSpec((tile, _HW), lambda i: (i, 0)),
            pl.BlockSpec((tile, _HW), lambda i: (i, 0)),
            pl.BlockSpec((tile, DE), lambda i: (i, 0)),
            pl.BlockSpec((DE, H), lambda i: (0, 0)),
            pl.BlockSpec((1, H), lambda i: (0, 0)),
            pl.BlockSpec((H, H), lambda i: (0, 0)),
            pl.BlockSpec((1, H), lambda i: (0, 0)),
        ],
        out_specs=pl.BlockSpec((tile, H), lambda i: (i, 0)),
        out_shape=jax.ShapeDtypeStruct((ec, H), jnp.float32),
    )(g1, g2, ea, wea, be1, we2, be2)


_SCH = 104
_HH = H // NC
_RPS = 624
_RTAIL = N - NS * _RPS
_ZROWS = 104


def _make_scatter(ec):
    eps = ec // NS
    sfull = eps // _SCH
    spair = sfull // 2
    stail = eps - sfull * _SCH

    def body(mij_hbm, row_hbm, agg_hbm,
             idx_s, buf_s0, buf_s1, idx_t, buf_t, zbuf, acc,
             sem_l0, sem_l1):
        c = lax.axis_index("c")
        s = lax.axis_index("s")

        def zrow(i, _):
            for j in range(_HH // 16):
                zbuf[i, pl.ds(j * 16, 16)] = jnp.zeros((16,), jnp.float32)
            return 0

        lax.fori_loop(0, _ZROWS, zrow, 0, unroll=False)
        for k in range(_RPS // _ZROWS):
            pltpu.sync_copy(zbuf, acc.at[pl.ds(s * _RPS + k * _ZROWS,
                                               _ZROWS)])

        @pl.when(s == NS - 1)
        def _zero_tail():
            pltpu.sync_copy(zbuf.at[pl.ds(0, _RTAIL)],
                            acc.at[pl.ds(NS * _RPS, _RTAIL)])

        plsc.subcore_barrier()

        base = s * eps
        col0 = c * _HH

        def load(k, slot_idx, slot_buf, sem):
            off = base + k * _SCH
            pltpu.async_copy(row_hbm.at[pl.ds(off, _SCH)], slot_idx, sem)
            pltpu.async_copy(mij_hbm.at[pl.ds(off, _SCH), pl.ds(col0, _HH)],
                             slot_buf, sem)

        def drain(k, slot_idx, slot_buf, sem):
            off = base + k * _SCH
            pltpu.make_async_copy(row_hbm.at[pl.ds(off, _SCH)], slot_idx,
                                  sem).wait()
            pltpu.make_async_copy(
                mij_hbm.at[pl.ds(off, _SCH), pl.ds(col0, _HH)], slot_buf,
                sem).wait()

        load(0, idx_s.at[0], buf_s0, sem_l0)
        load(1, idx_s.at[1], buf_s1, sem_l1)

        def pair(t, _):
            drain(2 * t, idx_s.at[0], buf_s0, sem_l0)
            pltpu.sync_copy(buf_s0, acc.at[idx_s.at[0]], add=True)

            @pl.when(t < spair - 1)
            def _next0():
                load(2 * t + 2, idx_s.at[0], buf_s0, sem_l0)

            drain(2 * t + 1, idx_s.at[1], buf_s1, sem_l1)
            pltpu.sync_copy(buf_s1, acc.at[idx_s.at[1]], add=True)

            @pl.when(t < spair - 1)
            def _next1():
                load(2 * t + 3, idx_s.at[1], buf_s1, sem_l1)

            return 0

        lax.fori_loop(0, spair, pair, 0, unroll=False)

        if stail:
            off = base + sfull * _SCH
            pltpu.sync_copy(row_hbm.at[pl.ds(off, stail)], idx_t)
            pltpu.sync_copy(mij_hbm.at[pl.ds(off, stail), pl.ds(col0, _HH)],
                            buf_t)
            pltpu.sync_copy(buf_t, acc.at[idx_t], add=True)

        plsc.subcore_barrier()

        r0 = s * _RPS
        pltpu.sync_copy(acc.at[pl.ds(r0, _RPS)],
                        agg_hbm.at[c, pl.ds(r0, _RPS)])

        @pl.when(s == NS - 1)
        def _write_tail():
            pltpu.sync_copy(acc.at[pl.ds(NS * _RPS, _RTAIL)],
                            agg_hbm.at[c, pl.ds(NS * _RPS, _RTAIL)])

    tl = max(stail, 8)
    mesh = plsc.VectorSubcoreMesh(core_axis_name="c", subcore_axis_name="s")
    return pl.kernel(
        body,
        out_type=jax.ShapeDtypeStruct((NC, N, _HH), jnp.float32),
        mesh=mesh,
        scratch_types=[
            pltpu.VMEM((2, _SCH), jnp.int32),
            pltpu.VMEM((_SCH, _HH), jnp.float32),
            pltpu.VMEM((_SCH, _HH), jnp.float32),
            pltpu.VMEM((tl,), jnp.int32),
            pltpu.VMEM((tl, _HH), jnp.float32),
            pltpu.VMEM((_ZROWS, _HH), jnp.float32),
            pltpu.VMEM_SHARED((N, _HH), jnp.float32),
            pltpu.SemaphoreType.DMA,
            pltpu.SemaphoreType.DMA,
        ],
    )


def _scatter_a(*args):
    return _make_scatter(_EA)(*args)


def _scatter_b(*args):
    return _make_scatter(_EB)(*args)


_NODE_TILE = 2000


def _node_body(h_ref, aa0_ref, aa1_ref, ab0_ref, ab1_ref,
               wa_ref, w0_ref, w1_ref, bn1_ref, wn2_ref, bn2_ref, out_ref):
    hb = h_ref[...]
    a0 = aa0_ref[0] + ab0_ref[0]
    a1 = aa1_ref[0] + ab1_ref[0]
    pre = (jnp.dot(hb, wa_ref[...], preferred_element_type=jnp.float32)
           + jnp.dot(a0, w0_ref[...], preferred_element_type=jnp.float32)
           + jnp.dot(a1, w1_ref[...], preferred_element_type=jnp.float32)
           + bn1_ref[...])
    y = pre * jax.nn.sigmoid(pre)
    out_ref[...] = hb + jnp.dot(y, wn2_ref[...],
                                preferred_element_type=jnp.float32) \
        + bn2_ref[...]


def _node_mlp(h, agg_a, agg_b, wa, w0, w1, bn1, wn2, bn2):
    grid = (N // _NODE_TILE,)
    return pl.pallas_call(
        _node_body,
        grid=grid,
        in_specs=[
            pl.BlockSpec((_NODE_TILE, D), lambda i: (i, 0)),
            pl.BlockSpec((1, _NODE_TILE, _HH), lambda i: (0, i, 0)),
            pl.BlockSpec((1, _NODE_TILE, _HH), lambda i: (1, i, 0)),
            pl.BlockSpec((1, _NODE_TILE, _HH), lambda i: (0, i, 0)),
            pl.BlockSpec((1, _NODE_TILE, _HH), lambda i: (1, i, 0)),
            pl.BlockSpec((D, H), lambda i: (0, 0)),
            pl.BlockSpec((_HH, H), lambda i: (0, 0)),
            pl.BlockSpec((_HH, H), lambda i: (0, 0)),
            pl.BlockSpec((1, H), lambda i: (0, 0)),
            pl.BlockSpec((H, D), lambda i: (0, 0)),
            pl.BlockSpec((1, D), lambda i: (0, 0)),
        ],
        out_specs=pl.BlockSpec((_NODE_TILE, D), lambda i: (i, 0)),
        out_shape=jax.ShapeDtypeStruct((N, D), jnp.float32),
    )(h, agg_a, agg_a, agg_b, agg_b, wa, w0, w1, bn1, wn2, bn2)


def kernel(h, edge_index, edge_attr, We1, be1, We2, be2, Wn1, bn1, Wn2, bn2):
    row = edge_index[0]
    col = edge_index[1]
    ws = We1[:D]
    wt = We1[D:2 * D]
    wea = We1[2 * D:]
    we2b = We2.astype(jnp.bfloat16)
    be1r = be1.reshape(1, H)
    be2r = be2.reshape(1, H)

    row_a, row_b = row[:_EA], row[_EA:]
    col_a, col_b = col[:_EA], col[_EA:]
    ea_a, ea_b = edge_attr[:_EA], edge_attr[_EA:]

    a32, b32 = _pre_pass(h, ws, wt)

    g1a, g2a = _gather_a(a32, b32, row_a, col_a)
    g1b, g2b = _gather_b(a32, b32, row_b, col_b)

    mij_a = _edge_mlp(g1a, g2a, ea_a, wea, be1r, we2b, be2r)
    mij_b = _edge_mlp(g1b, g2b, ea_b, wea, be1r, we2b, be2r)

    agg_a = _scatter_a(mij_a, row_a)
    agg_b = _scatter_b(mij_b, row_b)

    mij = jnp.concatenate([mij_a, mij_b], axis=0)

    h_out = _node_mlp(h, agg_a, agg_b, Wn1[:D], Wn1[D:D + _HH],
                      Wn1[D + _HW:], bn1.reshape(1, H), Wn2,
                      bn2.reshape(1, D))
    return (h_out, mij)


# speedup vs baseline: 3.6760x; 1.0022x over previous
import jax
import jax.numpy as jnp
from jax import lax
from jax.experimental import pallas as pl
from jax.experimental.pallas import tpu as pltpu
from jax.experimental.pallas import tpu_sc as plsc

N = 10000
E = 160000
D = 256
H = 256
DE = 16

NC = 2
NS = 16
NW = NC * NS

_HW = H // 2
_GCH = 104

_EA = 80128
_EB = E - _EA


_PRE_TILE = 2000


def _pack_halves(x):
    xb = x.astype(jnp.bfloat16)
    lo = lax.bitcast_convert_type(xb[:, :_HW], jnp.uint16).astype(jnp.uint32)
    hi = lax.bitcast_convert_type(xb[:, _HW:], jnp.uint16).astype(jnp.uint32)
    return lax.bitcast_convert_type((hi << 16) | lo, jnp.int32)


def _pre_body(h_ref, ws_ref, wt_ref, a_ref, b_ref):
    hb = h_ref[...]
    a_ref[...] = _pack_halves(
        jnp.dot(hb, ws_ref[...], preferred_element_type=jnp.float32))
    b_ref[...] = _pack_halves(
        jnp.dot(hb, wt_ref[...], preferred_element_type=jnp.float32))


def _pre_pass(h, ws, wt):
    grid = (N // _PRE_TILE,)
    return pl.pallas_call(
        _pre_body,
        grid=grid,
        in_specs=[
            pl.BlockSpec((_PRE_TILE, D), lambda i: (i, 0)),
            pl.BlockSpec((D, H), lambda i: (0, 0)),
            pl.BlockSpec((D, H), lambda i: (0, 0)),
        ],
        out_specs=[
            pl.BlockSpec((_PRE_TILE, _HW), lambda i: (i, 0)),
            pl.BlockSpec((_PRE_TILE, _HW), lambda i: (i, 0)),
        ],
        out_shape=[
            jax.ShapeDtypeStruct((N, _HW), jnp.int32),
            jax.ShapeDtypeStruct((N, _HW), jnp.int32),
        ],
    )(h, ws, wt)


def _make_gather(ec):
    epw = ec // NW
    gfull = epw // _GCH
    gpair = gfull // 2
    gtail = epw - gfull * _GCH

    def body(a_hbm, b_hbm, row_hbm, col_hbm, g1_hbm, g2_hbm,
             idx_r, idx_c, buf_a0, buf_b0, buf_a1, buf_b1,
             buf_at, buf_bt,
             sem_i, sem_a0, sem_b0, sem_a1, sem_b1, sem_w0, sem_w1):
        c = lax.axis_index("c")
        s = lax.axis_index("s")
        wid = c * NS + s
        base = wid * epw

        ci_r = pltpu.async_copy(row_hbm.at[pl.ds(base, epw)], idx_r, sem_i)
        ci_c = pltpu.async_copy(col_hbm.at[pl.ds(base, epw)], idx_c, sem_i)
        ci_r.wait()
        ci_c.wait()

        def pair(t, _):
            off0 = base + (2 * t) * _GCH
            off1 = off0 + _GCH
            k0 = pl.multiple_of((2 * t) * _GCH, 8)
            k1 = pl.multiple_of((2 * t + 1) * _GCH, 8)

            @pl.when(t > 0)
            def _drain0():
                prev = off0 - 2 * _GCH
                pltpu.make_async_copy(
                    buf_a0, g1_hbm.at[pl.ds(prev, _GCH)], sem_w0).wait()
                pltpu.make_async_copy(
                    buf_b0, g2_hbm.at[pl.ds(prev, _GCH)], sem_w0).wait()

            cp_a0 = pltpu.async_copy(a_hbm.at[idx_r.at[pl.ds(k0, _GCH)]],
                                     buf_a0, sem_a0)
            cp_b0 = pltpu.async_copy(b_hbm.at[idx_c.at[pl.ds(k0, _GCH)]],
                                     buf_b0, sem_b0)

            @pl.when(t > 0)
            def _drain1():
                prev = off1 - 2 * _GCH
                pltpu.make_async_copy(
                    buf_a1, g1_hbm.at[pl.ds(prev, _GCH)], sem_w1).wait()
                pltpu.make_async_copy(
                    buf_b1, g2_hbm.at[pl.ds(prev, _GCH)], sem_w1).wait()

            cp_a1 = pltpu.async_copy(a_hbm.at[idx_r.at[pl.ds(k1, _GCH)]],
                                     buf_a1, sem_a1)
            cp_b1 = pltpu.async_copy(b_hbm.at[idx_c.at[pl.ds(k1, _GCH)]],
                                     buf_b1, sem_b1)

            cp_a0.wait()
            cp_b0.wait()
            pltpu.async_copy(buf_a0, g1_hbm.at[pl.ds(off0, _GCH)], sem_w0)
            pltpu.async_copy(buf_b0, g2_hbm.at[pl.ds(off0, _GCH)], sem_w0)

            cp_a1.wait()
            cp_b1.wait()
            pltpu.async_copy(buf_a1, g1_hbm.at[pl.ds(off1, _GCH)], sem_w1)
            pltpu.async_copy(buf_b1, g2_hbm.at[pl.ds(off1, _GCH)], sem_w1)
            return 0

        lax.fori_loop(0, gpair, pair, 0, unroll=False)

        last0 = base + (gfull - 2) * _GCH
        last1 = last0 + _GCH
        pltpu.make_async_copy(buf_a0, g1_hbm.at[pl.ds(last0, _GCH)],
                              sem_w0).wait()
        pltpu.make_async_copy(buf_b0, g2_hbm.at[pl.ds(last0, _GCH)],
                              sem_w0).wait()
        pltpu.make_async_copy(buf_a1, g1_hbm.at[pl.ds(last1, _GCH)],
                              sem_w1).wait()
        pltpu.make_async_copy(buf_b1, g2_hbm.at[pl.ds(last1, _GCH)],
                              sem_w1).wait()

        if gtail:
            off = base + gfull * _GCH
            kt = gfull * _GCH
            cp_a = pltpu.async_copy(a_hbm.at[idx_r.at[pl.ds(kt, gtail)]],
                                    buf_at, sem_a0)
            cp_b = pltpu.async_copy(b_hbm.at[idx_c.at[pl.ds(kt, gtail)]],
                                    buf_bt, sem_b0)
            cp_a.wait()
            cp_b.wait()
            pltpu.sync_copy(buf_at, g1_hbm.at[pl.ds(off, gtail)])
            pltpu.sync_copy(buf_bt, g2_hbm.at[pl.ds(off, gtail)])

    tl = max(gtail, 8)
    mesh = plsc.VectorSubcoreMesh(core_axis_name="c", subcore_axis_name="s")
    return pl.kernel(
        body,
        out_type=[
            jax.ShapeDtypeStruct((ec, _HW), jnp.int32),
            jax.ShapeDtypeStruct((ec, _HW), jnp.int32),
        ],
        mesh=mesh,
        scratch_types=[
            pltpu.VMEM((epw,), jnp.int32),
            pltpu.VMEM((epw,), jnp.int32),
            pltpu.VMEM((_GCH, _HW), jnp.int32),
            pltpu.VMEM((_GCH, _HW), jnp.int32),
            pltpu.VMEM((_GCH, _HW), jnp.int32),
            pltpu.VMEM((_GCH, _HW), jnp.int32),
            pltpu.VMEM((tl, _HW), jnp.int32),
            pltpu.VMEM((tl, _HW), jnp.int32),
            pltpu.SemaphoreType.DMA,
            pltpu.SemaphoreType.DMA,
            pltpu.SemaphoreType.DMA,
            pltpu.SemaphoreType.DMA,
            pltpu.SemaphoreType.DMA,
            pltpu.SemaphoreType.DMA,
            pltpu.SemaphoreType.DMA,
        ],
    )


def _gather_a(*args):
    return _make_gather(_EA)(*args)


def _gather_b(*args):
    return _make_gather(_EB)(*args)


def _unpack_halves(g32):
    gu = lax.bitcast_convert_type(g32, jnp.uint32)
    lo = lax.bitcast_convert_type((gu & 0xFFFF).astype(jnp.uint16),
                                  jnp.bfloat16).astype(jnp.float32)
    hi = lax.bitcast_convert_type((gu >> 16).astype(jnp.uint16),
                                  jnp.bfloat16).astype(jnp.float32)
    return lo, hi


def _edge_body(g1_ref, g2_ref, ea_ref, wea_ref, be1_ref, we2_ref, be2_ref,
               out_ref):
    g1_lo, g1_hi = _unpack_halves(g1_ref[...])
    g2_lo, g2_hi = _unpack_halves(g2_ref[...])
    ea = ea_ref[...]
    eam = jnp.dot(ea, wea_ref[...], preferred_element_type=jnp.float32)
    pre_lo = g1_lo + g2_lo + eam[:, :_HW] + be1_ref[:, :_HW]
    pre_hi = g1_hi + g2_hi + eam[:, _HW:] + be1_ref[:, _HW:]
    m_lo = pre_lo * jax.nn.sigmoid(pre_lo)
    m_hi = pre_hi * jax.nn.sigmoid(pre_hi)
    pre2 = (jnp.dot(m_lo.astype(jnp.bfloat16), we2_ref[:_HW],
                    preferred_element_type=jnp.float32)
            + jnp.dot(m_hi.astype(jnp.bfloat16), we2_ref[_HW:],
                      preferred_element_type=jnp.float32)
            + be2_ref[...])
    out_ref[...] = pre2 * jax.nn.sigmoid(pre2)


def _edge_mlp(g1, g2, ea, wea, be1, we2, be2):
    ec = g1.shape[0]
    tile = ec // 32
    return pl.pallas_call(
        _edge_body,
        grid=(32,),
        in_specs=[
            pl.BlockSpec((tile, _HW), lambda i: (i, 0)),
            pl.BlockSpec((tile, _HW), lambda i: (i, 0)),
            pl.BlockSpec((tile, DE), lambda i: (i, 0)),
            pl.BlockSpec((DE, H), lambda i: (0, 0)),
            pl.BlockSpec((1, H), lambda i: (0, 0)),
            pl.BlockSpec((H, H), lambda i: (0, 0)),
            pl.BlockSpec((1, H), lambda i: (0, 0)),
        ],
        out_specs=pl.BlockSpec((tile, H), lambda i: (i, 0)),
        out_shape=jax.ShapeDtypeStruct((ec, H), jnp.float32),
    )(g1, g2, ea, wea, be1, we2, be2)


_SCH = 104
_HH = H // NC
_RPS = 624
_RTAIL = N - NS * _RPS
_ZROWS = 104


def _make_scatter(ec):
    eps = ec // NS
    sfull = eps // _SCH
    spair = sfull // 2
    stail = eps - sfull * _SCH

    def body(mij_hbm, row_hbm, agg_hbm,
             idx_s, buf_s0, buf_s1, idx_t, buf_t, zbuf, acc,
             sem_l0, sem_l1, sem_c0, sem_c1):
        c = lax.axis_index("c")
        s = lax.axis_index("s")

        def zrow(i, _):
            for j in range(_HH // 16):
                zbuf[i, pl.ds(j * 16, 16)] = jnp.zeros((16,), jnp.float32)
            return 0

        lax.fori_loop(0, _ZROWS, zrow, 0, unroll=False)
        for k in range(_RPS // _ZROWS):
            pltpu.sync_copy(zbuf, acc.at[pl.ds(s * _RPS + k * _ZROWS,
                                               _ZROWS)])

        @pl.when(s == NS - 1)
        def _zero_tail():
            pltpu.sync_copy(zbuf.at[pl.ds(0, _RTAIL)],
                            acc.at[pl.ds(NS * _RPS, _RTAIL)])

        plsc.subcore_barrier()

        base = s * eps
        col0 = c * _HH

        def load(k, slot_idx, slot_buf, sem):
            off = base + k * _SCH
            pltpu.async_copy(row_hbm.at[pl.ds(off, _SCH)], slot_idx, sem)
            pltpu.async_copy(mij_hbm.at[pl.ds(off, _SCH), pl.ds(col0, _HH)],
                             slot_buf, sem)

        def drain(k, slot_idx, slot_buf, sem):
            off = base + k * _SCH
            pltpu.make_async_copy(row_hbm.at[pl.ds(off, _SCH)], slot_idx,
                                  sem).wait()
            pltpu.make_async_copy(
                mij_hbm.at[pl.ds(off, _SCH), pl.ds(col0, _HH)], slot_buf,
                sem).wait()

        load(0, idx_s.at[0], buf_s0, sem_l0)
        load(1, idx_s.at[1], buf_s1, sem_l1)

        def pair(t, _):
            drain(2 * t, idx_s.at[0], buf_s0, sem_l0)
            pltpu.async_copy(buf_s0, acc.at[idx_s.at[0]], sem_c0, add=True)

            drain(2 * t + 1, idx_s.at[1], buf_s1, sem_l1)
            pltpu.async_copy(buf_s1, acc.at[idx_s.at[1]], sem_c1, add=True)

            @pl.when(t < spair - 1)
            def _next0():
                pltpu.make_async_copy(buf_s0, acc.at[idx_s.at[0]],
                                      sem_c0).wait()
                load(2 * t + 2, idx_s.at[0], buf_s0, sem_l0)

            @pl.when(t < spair - 1)
            def _next1():
                pltpu.make_async_copy(buf_s1, acc.at[idx_s.at[1]],
                                      sem_c1).wait()
                load(2 * t + 3, idx_s.at[1], buf_s1, sem_l1)

            return 0

        lax.fori_loop(0, spair, pair, 0, unroll=False)
        pltpu.make_async_copy(buf_s0, acc.at[idx_s.at[0]], sem_c0).wait()
        pltpu.make_async_copy(buf_s1, acc.at[idx_s.at[1]], sem_c1).wait()

        if stail:
            off = base + sfull * _SCH
            pltpu.sync_copy(row_hbm.at[pl.ds(off, stail)], idx_t)
            pltpu.sync_copy(mij_hbm.at[pl.ds(off, stail), pl.ds(col0, _HH)],
                            buf_t)
            pltpu.sync_copy(buf_t, acc.at[idx_t], add=True)

        plsc.subcore_barrier()

        r0 = s * _RPS
        pltpu.sync_copy(acc.at[pl.ds(r0, _RPS)],
                        agg_hbm.at[c, pl.ds(r0, _RPS)])

        @pl.when(s == NS - 1)
        def _write_tail():
            pltpu.sync_copy(acc.at[pl.ds(NS * _RPS, _RTAIL)],
                            agg_hbm.at[c, pl.ds(NS * _RPS, _RTAIL)])

    tl = max(stail, 8)
    mesh = plsc.VectorSubcoreMesh(core_axis_name="c", subcore_axis_name="s")
    return pl.kernel(
        body,
        out_type=jax.ShapeDtypeStruct((NC, N, _HH), jnp.float32),
        mesh=mesh,
        scratch_types=[
            pltpu.VMEM((2, _SCH), jnp.int32),
            pltpu.VMEM((_SCH, _HH), jnp.float32),
            pltpu.VMEM((_SCH, _HH), jnp.float32),
            pltpu.VMEM((tl,), jnp.int32),
            pltpu.VMEM((tl, _HH), jnp.float32),
            pltpu.VMEM((_ZROWS, _HH), jnp.float32),
            pltpu.VMEM_SHARED((N, _HH), jnp.float32),
            pltpu.SemaphoreType.DMA,
            pltpu.SemaphoreType.DMA,
            pltpu.SemaphoreType.DMA,
            pltpu.SemaphoreType.DMA,
        ],
    )


def _scatter_a(*args):
    return _make_scatter(_EA)(*args)


def _scatter_b(*args):
    return _make_scatter(_EB)(*args)


_NODE_TILE = 2000


def _node_body(h_ref, aa0_ref, aa1_ref, ab0_ref, ab1_ref,
               wa_ref, w0_ref, w1_ref, bn1_ref, wn2_ref, bn2_ref, out_ref):
    hb = h_ref[...]
    a0 = aa0_ref[0] + ab0_ref[0]
    a1 = aa1_ref[0] + ab1_ref[0]
    pre = (jnp.dot(hb, wa_ref[...], preferred_element_type=jnp.float32)
           + jnp.dot(a0, w0_ref[...], preferred_element_type=jnp.float32)
           + jnp.dot(a1, w1_ref[...], preferred_element_type=jnp.float32)
           + bn1_ref[...])
    y = pre * jax.nn.sigmoid(pre)
    out_ref[...] = hb + jnp.dot(y, wn2_ref[...],
                                preferred_element_type=jnp.float32) \
        + bn2_ref[...]


def _node_mlp(h, agg_a, agg_b, wa, w0, w1, bn1, wn2, bn2):
    grid = (N // _NODE_TILE,)
    return pl.pallas_call(
        _node_body,
        grid=grid,
        in_specs=[
            pl.BlockSpec((_NODE_TILE, D), lambda i: (i, 0)),
            pl.BlockSpec((1, _NODE_TILE, _HH), lambda i: (0, i, 0)),
            pl.BlockSpec((1, _NODE_TILE, _HH), lambda i: (1, i, 0)),
            pl.BlockSpec((1, _NODE_TILE, _HH), lambda i: (0, i, 0)),
            pl.BlockSpec((1, _NODE_TILE, _HH), lambda i: (1, i, 0)),
            pl.BlockSpec((D, H), lambda i: (0, 0)),
            pl.BlockSpec((_HH, H), lambda i: (0, 0)),
            pl.BlockSpec((_HH, H), lambda i: (0, 0)),
            pl.BlockSpec((1, H), lambda i: (0, 0)),
            pl.BlockSpec((H, D), lambda i: (0, 0)),
            pl.BlockSpec((1, D), lambda i: (0, 0)),
        ],
        out_specs=pl.BlockSpec((_NODE_TILE, D), lambda i: (i, 0)),
        out_shape=jax.ShapeDtypeStruct((N, D), jnp.float32),
    )(h, agg_a, agg_a, agg_b, agg_b, wa, w0, w1, bn1, wn2, bn2)


def kernel(h, edge_index, edge_attr, We1, be1, We2, be2, Wn1, bn1, Wn2, bn2):
    row = edge_index[0]
    col = edge_index[1]
    ws = We1[:D]
    wt = We1[D:2 * D]
    wea = We1[2 * D:]
    we2b = We2.astype(jnp.bfloat16)
    be1r = be1.reshape(1, H)
    be2r = be2.reshape(1, H)

    row_a, row_b = row[:_EA], row[_EA:]
    col_a, col_b = col[:_EA], col[_EA:]
    ea_a, ea_b = edge_attr[:_EA], edge_attr[_EA:]

    a32, b32 = _pre_pass(h, ws, wt)

    g1a, g2a = _gather_a(a32, b32, row_a, col_a)
    g1b, g2b = _gather_b(a32, b32, row_b, col_b)

    mij_a = _edge_mlp(g1a, g2a, ea_a, wea, be1r, we2b, be2r)
    mij_b = _edge_mlp(g1b, g2b, ea_b, wea, be1r, we2b, be2r)

    agg_a = _scatter_a(mij_a, row_a)
    agg_b = _scatter_b(mij_b, row_b)

    mij = jnp.concatenate([mij_a, mij_b], axis=0)

    h_out = _node_mlp(h, agg_a, agg_b, Wn1[:D], Wn1[D:D + _HH],
                      Wn1[D + _HW:], bn1.reshape(1, H), Wn2,
                      bn2.reshape(1, D))
    return (h_out, mij)
